# Initial kernel scaffold; baseline (speedup 1.0000x reference)
import functools

import jax
import jax.numpy as jnp
from jax import lax
from jax.experimental import pallas as pl
from jax.experimental.pallas import tpu as pltpu
from jax.experimental.pallas import tpu_sc as plsc

NC = 2
NS = 16
L = 16
CH = 128

_MESH = dict(core_axis_name="c", subcore_axis_name="s")


def _rsqrt16(x):
    i = plsc.bitcast(x, jnp.int32)
    i = jnp.int32(0x5F3759DF) - (i >> 1)
    y = plsc.bitcast(i, jnp.float32)
    for _ in range(3):
        y = y * (1.5 - 0.5 * x * y * y)
    return y


def _make_norm_kernel(E_pad, N_pad):
    NPT = N_pad // NS
    EPT = E_pad // NS
    EPT2 = E_pad // (NS * NC)

    @functools.partial(
        pl.kernel,
        out_type=jax.ShapeDtypeStruct((E_pad,), jnp.float32),
        mesh=plsc.VectorSubcoreMesh(**_MESH),
        scratch_types=[
            pltpu.VMEM_SHARED((N_pad,), jnp.float32),
            pltpu.VMEM_SHARED((N_pad,), jnp.float32),
            pltpu.VMEM((N_pad,), jnp.float32),
            pltpu.VMEM((NPT,), jnp.float32),
            pltpu.VMEM((CH,), jnp.int32),
            pltpu.VMEM((CH,), jnp.int32),
            pltpu.VMEM((CH,), jnp.float32),
            pltpu.VMEM((CH,), jnp.float32),
        ],
    )
    def norm_kernel(src, dst, ew, norm_out,
                    deg_sp, dinv_sp, dinv_loc, slcb, ia, ib, vb, nb):
        c = lax.axis_index("c")
        s = lax.axis_index("s")
        z16 = jnp.zeros((L,), jnp.float32)

        def zero_body(k, _):
            slcb[pl.ds(k * L, L)] = z16
            return 0
        lax.fori_loop(0, NPT // L, zero_body, 0)
        pltpu.sync_copy(slcb, deg_sp.at[pl.ds(s * NPT, NPT)])
        plsc.subcore_barrier()

        def deg_body(k, _):
            b = s * EPT + k * CH
            pltpu.sync_copy(dst.at[pl.ds(b, CH)], ia)
            pltpu.sync_copy(ew.at[pl.ds(b, CH)], vb)
            pltpu.sync_copy(vb, deg_sp.at[ia], add=True)
            return 0
        lax.fori_loop(0, EPT // CH, deg_body, 0)
        plsc.subcore_barrier()

        pltpu.sync_copy(deg_sp.at[pl.ds(s * NPT, NPT)], slcb)
        def rsq_body(k, _):
            v = slcb[pl.ds(k * L, L)]
            slcb[pl.ds(k * L, L)] = _rsqrt16(v)
            return 0
        lax.fori_loop(0, NPT // L, rsq_body, 0)
        pltpu.sync_copy(slcb, dinv_sp.at[pl.ds(s * NPT, NPT)])
        plsc.subcore_barrier()

        pltpu.sync_copy(dinv_sp, dinv_loc)

        w = c * NS + s
        def nrm_body(k, _):
            b = w * EPT2 + k * CH
            pltpu.sync_copy(src.at[pl.ds(b, CH)], ia)
            pltpu.sync_copy(dst.at[pl.ds(b, CH)], ib)
            pltpu.sync_copy(ew.at[pl.ds(b, CH)], vb)
            for j in range(CH // L):
                s16 = ia[pl.ds(j * L, L)]
                d16 = ib[pl.ds(j * L, L)]
                e16 = vb[pl.ds(j * L, L)]
                a = plsc.load_gather(dinv_loc, [s16])
                bgt = plsc.load_gather(dinv_loc, [d16])
                nb[pl.ds(j * L, L)] = a * e16 * bgt
            pltpu.sync_copy(nb, norm_out.at[pl.ds(b, CH)])
            return 0
        lax.fori_loop(0, EPT2 // CH, nrm_body, 0)

    return norm_kernel


def _make_agg1_kernel(E_pad, N_pad, F):
    NPT = N_pad // NS
    EPT = E_pad // NS

    @functools.partial(
        pl.kernel,
        out_type=(jax.ShapeDtypeStruct((N_pad, F), jnp.float32),
                  jax.ShapeDtypeStruct((N_pad, F), jnp.float32)),
        mesh=plsc.VectorSubcoreMesh(**_MESH),
        scratch_types=[
            pltpu.VMEM_SHARED((N_pad, F), jnp.float32),
            pltpu.VMEM((L, F), jnp.float32),
            pltpu.VMEM((CH,), jnp.int32),
            pltpu.VMEM((CH,), jnp.int32),
            pltpu.VMEM((CH,), jnp.float32),
            pltpu.VMEM((CH, F), jnp.float32),
        ],
    )
    def agg1_kernel(src, dst, norm, x0, x1, out0, out1,
                    acc, zb, sidx, didx, nb, rows):
        c = lax.axis_index("c")
        s = lax.axis_index("s")
        z16 = jnp.zeros((L,), jnp.float32)

        def body(xc, outc):
            for r in range(L):
                for j in range(F // L):
                    zb[r, pl.ds(j * L, L)] = z16
            def zcp(k, _):
                pltpu.sync_copy(zb, acc.at[pl.ds(s * NPT + k * L, L)])
                return 0
            lax.fori_loop(0, NPT // L, zcp, 0)
            plsc.subcore_barrier()

            def chunk(k, _):
                b = s * EPT + k * CH
                pltpu.sync_copy(src.at[pl.ds(b, CH)], sidx)
                pltpu.sync_copy(dst.at[pl.ds(b, CH)], didx)
                pltpu.sync_copy(norm.at[pl.ds(b, CH)], nb)
                pltpu.sync_copy(xc.at[sidx], rows)
                def row(r, _):
                    v16 = jnp.full((L,), nb[r], jnp.float32)
                    for j in range(F // L):
                        rows[r, pl.ds(j * L, L)] = rows[r, pl.ds(j * L, L)] * v16
                    return 0
                lax.fori_loop(0, CH, row, 0)
                pltpu.sync_copy(rows, acc.at[didx], add=True)
                return 0
            lax.fori_loop(0, EPT // CH, chunk, 0)
            plsc.subcore_barrier()

            pltpu.sync_copy(acc.at[pl.ds(s * NPT, NPT)],
                            outc.at[pl.ds(s * NPT, NPT)])

        @pl.when(c == 0)
        def _():
            body(x0, out0)

        @pl.when(c == 1)
        def _():
            body(x1, out1)

    return agg1_kernel


def _make_agg2_kernel(E_pad, N_pad):
    NPT = N_pad // NS
    EPT = E_pad // NS

    @functools.partial(
        pl.kernel,
        out_type=jax.ShapeDtypeStruct((2, N_pad), jnp.float32),
        mesh=plsc.VectorSubcoreMesh(**_MESH),
        scratch_types=[
            pltpu.VMEM_SHARED((N_pad,), jnp.float32),
            pltpu.VMEM_SHARED((N_pad,), jnp.float32),
            pltpu.VMEM((N_pad // NS,), jnp.float32),
            pltpu.VMEM((CH,), jnp.int32),
            pltpu.VMEM((CH,), jnp.int32),
            pltpu.VMEM((CH,), jnp.float32),
            pltpu.VMEM((CH,), jnp.float32),
            pltpu.VMEM((CH,), jnp.float32),
        ],
    )
    def agg2_kernel(src, dst, norm, h0, h1, pout,
                    acc0, acc1, zb, sidx, didx, nb, g0, g1):
        c = lax.axis_index("c")
        s = lax.axis_index("s")
        z16 = jnp.zeros((L,), jnp.float32)

        @pl.when(c == 0)
        def _():
            def zero_body(k, _):
                zb[pl.ds(k * L, L)] = z16
                return 0
            lax.fori_loop(0, NPT // L, zero_body, 0)
            pltpu.sync_copy(zb, acc0.at[pl.ds(s * NPT, NPT)])
            pltpu.sync_copy(zb, acc1.at[pl.ds(s * NPT, NPT)])
            plsc.subcore_barrier()

            def chunk(k, _):
                b = s * EPT + k * CH
                pltpu.sync_copy(src.at[pl.ds(b, CH)], sidx)
                pltpu.sync_copy(dst.at[pl.ds(b, CH)], didx)
                pltpu.sync_copy(norm.at[pl.ds(b, CH)], nb)
                pltpu.sync_copy(h0.at[sidx], g0)
                pltpu.sync_copy(h1.at[sidx], g1)
                for j in range(CH // L):
                    sl = pl.ds(j * L, L)
                    n16 = nb[sl]
                    g0[sl] = g0[sl] * n16
                    g1[sl] = g1[sl] * n16
                pltpu.sync_copy(g0, acc0.at[didx], add=True)
                pltpu.sync_copy(g1, acc1.at[didx], add=True)
                return 0
            lax.fori_loop(0, EPT // CH, chunk, 0)
            plsc.subcore_barrier()

            pltpu.sync_copy(acc0.at[pl.ds(s * NPT, NPT)],
                            pout.at[0, pl.ds(s * NPT, NPT)])
            pltpu.sync_copy(acc1.at[pl.ds(s * NPT, NPT)],
                            pout.at[1, pl.ds(s * NPT, NPT)])

    return agg2_kernel


def _make_mlp_kernel(N_pad, D, H, OUT):
    BR = 256
    F = D // 2

    def body(a0, a1, w1, b1r, w2, o):
        w1v = w1[...]
        h = (jnp.dot(a0[...], w1v[:F, :], preferred_element_type=jnp.float32)
             + jnp.dot(a1[...], w1v[F:, :], preferred_element_type=jnp.float32)
             + b1r[...])
        h = jnp.maximum(h, 0.0)
        o[...] = jnp.dot(h, w2[...], preferred_element_type=jnp.float32)

    return pl.pallas_call(
        body,
        grid=(N_pad // BR,),
        in_specs=[
            pl.BlockSpec((BR, F), lambda i: (i, 0)),
            pl.BlockSpec((BR, F), lambda i: (i, 0)),
            pl.BlockSpec((D, H), lambda i: (0, 0)),
            pl.BlockSpec((1, H), lambda i: (0, 0)),
            pl.BlockSpec((H, OUT), lambda i: (0, 0)),
        ],
        out_specs=pl.BlockSpec((BR, OUT), lambda i: (i, 0)),
        out_shape=jax.ShapeDtypeStruct((N_pad, OUT), jnp.float32),
    )


def kernel(x, edge_index, edge_weight, W1, b1, W2, b2):
    N, D = x.shape
    E = edge_index.shape[1]
    H = W1.shape[1]
    OUT = W2.shape[1]

    N_pad = -(-N // (NS * L)) * (NS * L)
    E1 = E + N
    step = NS * NC * CH
    E_pad = -(-E1 // step) * step

    src = edge_index[0].astype(jnp.int32)
    dst = edge_index[1].astype(jnp.int32)
    loop = jnp.arange(N, dtype=jnp.int32)
    padz = jnp.zeros((E_pad - E1,), jnp.int32)
    srcp = jnp.concatenate([src, loop, padz])
    dstp = jnp.concatenate([dst, loop, padz])
    ewp = jnp.concatenate([edge_weight.astype(jnp.float32),
                           jnp.ones((N,), jnp.float32),
                           jnp.zeros((E_pad - E1,), jnp.float32)])

    xp = jnp.pad(x.astype(jnp.float32), ((0, N_pad - N), (0, 0)))
    x0 = xp[:, : D // 2]
    x1 = xp[:, D // 2:]

    norm = _make_norm_kernel(E_pad, N_pad)(srcp, dstp, ewp)
    agg0, agg1 = _make_agg1_kernel(E_pad, N_pad, D // 2)(srcp, dstp, norm, x0, x1)
    hw2 = _make_mlp_kernel(N_pad, D, H, OUT)(agg0, agg1, W1, b1.reshape(1, H), W2)
    h0 = hw2[:, 0]
    h1 = hw2[:, 1]
    pout = _make_agg2_kernel(E_pad, N_pad)(srcp, dstp, norm, h0, h1)
    out = jnp.stack([pout[0, :N], pout[1, :N]], axis=1) + b2
    return out


# speedup vs baseline: 5.4599x; 5.4599x over previous
import functools

import jax
import jax.numpy as jnp
from jax import lax
from jax.experimental import pallas as pl
from jax.experimental.pallas import tpu as pltpu
from jax.experimental.pallas import tpu_sc as plsc

NC = 2
NS = 16
L = 16
CH = 128

_MESH = dict(core_axis_name="c", subcore_axis_name="s")


def _rsqrt16(x):
    i = lax.bitcast_convert_type(x, jnp.int32)
    i = jnp.int32(0x5F3759DF) - (i >> 1)
    y = lax.bitcast_convert_type(i, jnp.float32)
    for _ in range(3):
        y = y * (1.5 - 0.5 * x * y * y)
    return y


def _make_norm_kernel(E_pad, N_pad):
    NPT = N_pad // NS
    EPT = E_pad // NS
    EPT2 = E_pad // (NS * NC)

    @functools.partial(
        pl.kernel,
        out_type=jax.ShapeDtypeStruct((E_pad,), jnp.float32),
        mesh=plsc.VectorSubcoreMesh(**_MESH),
        compiler_params=pltpu.CompilerParams(needs_layout_passes=False),
        scratch_types=[
            pltpu.VMEM_SHARED((N_pad,), jnp.float32),
            pltpu.VMEM_SHARED((N_pad,), jnp.float32),
            pltpu.VMEM((N_pad,), jnp.float32),
            pltpu.VMEM((NPT,), jnp.float32),
            pltpu.VMEM((CH,), jnp.int32),
            pltpu.VMEM((CH,), jnp.int32),
            pltpu.VMEM((CH,), jnp.float32),
            pltpu.VMEM((CH,), jnp.float32),
        ],
    )
    def norm_kernel(src, dst, ew, norm_out,
                    deg_sp, dinv_sp, dinv_loc, slcb, ia, ib, vb, nb):
        c = lax.axis_index("c")
        s = lax.axis_index("s")
        z16 = jnp.zeros((L,), jnp.float32)

        def zero_body(k, _):
            slcb[pl.ds(k * L, L)] = z16
            return 0
        lax.fori_loop(0, NPT // L, zero_body, 0)
        pltpu.sync_copy(slcb, deg_sp.at[pl.ds(s * NPT, NPT)])
        plsc.subcore_barrier()

        def deg_body(k, _):
            b = s * EPT + k * CH
            pltpu.sync_copy(dst.at[pl.ds(b, CH)], ia)
            pltpu.sync_copy(ew.at[pl.ds(b, CH)], vb)
            pltpu.sync_copy(vb, deg_sp.at[ia], add=True)
            return 0
        lax.fori_loop(0, EPT // CH, deg_body, 0)
        plsc.subcore_barrier()

        pltpu.sync_copy(deg_sp.at[pl.ds(s * NPT, NPT)], slcb)
        def rsq_body(k, _):
            v = slcb[pl.ds(k * L, L)]
            slcb[pl.ds(k * L, L)] = _rsqrt16(v)
            return 0
        lax.fori_loop(0, NPT // L, rsq_body, 0)
        pltpu.sync_copy(slcb, dinv_sp.at[pl.ds(s * NPT, NPT)])
        plsc.subcore_barrier()

        pltpu.sync_copy(dinv_sp, dinv_loc)

        w = c * NS + s
        def nrm_body(k, _):
            b = w * EPT2 + k * CH
            pltpu.sync_copy(src.at[pl.ds(b, CH)], ia)
            pltpu.sync_copy(dst.at[pl.ds(b, CH)], ib)
            pltpu.sync_copy(ew.at[pl.ds(b, CH)], vb)
            for j in range(CH // L):
                s16 = ia[pl.ds(j * L, L)]
                d16 = ib[pl.ds(j * L, L)]
                e16 = vb[pl.ds(j * L, L)]
                a = plsc.load_gather(dinv_loc, [s16])
                bgt = plsc.load_gather(dinv_loc, [d16])
                nb[pl.ds(j * L, L)] = a * e16 * bgt
            pltpu.sync_copy(nb, norm_out.at[pl.ds(b, CH)])
            return 0
        lax.fori_loop(0, EPT2 // CH, nrm_body, 0)

    return norm_kernel


def _make_agg1_kernel(E_pad, N_pad, F):
    NPT = N_pad // NS
    EPT = E_pad // NS

    @functools.partial(
        pl.kernel,
        out_type=(jax.ShapeDtypeStruct((N_pad, F), jnp.float32),
                  jax.ShapeDtypeStruct((N_pad, F), jnp.float32)),
        mesh=plsc.VectorSubcoreMesh(**_MESH),
        compiler_params=pltpu.CompilerParams(needs_layout_passes=False),
        scratch_types=[
            pltpu.VMEM_SHARED((N_pad, F), jnp.float32),
            pltpu.VMEM((L, F), jnp.float32),
            pltpu.VMEM((CH,), jnp.int32),
            pltpu.VMEM((CH,), jnp.int32),
            pltpu.VMEM((CH,), jnp.float32),
            pltpu.VMEM((CH, F), jnp.float32),
        ],
    )
    def agg1_kernel(src, dst, norm, x0, x1, out0, out1,
                    acc, zb, sidx, didx, nb, rows):
        c = lax.axis_index("c")
        s = lax.axis_index("s")
        z16 = jnp.zeros((L,), jnp.float32)

        def body(xc, outc):
            for r in range(L):
                for j in range(F // L):
                    zb[r, pl.ds(j * L, L)] = z16
            def zcp(k, _):
                pltpu.sync_copy(zb, acc.at[pl.ds(s * NPT + k * L, L)])
                return 0
            lax.fori_loop(0, NPT // L, zcp, 0)
            plsc.subcore_barrier()

            def chunk(k, _):
                b = s * EPT + k * CH
                pltpu.sync_copy(src.at[pl.ds(b, CH)], sidx)
                pltpu.sync_copy(dst.at[pl.ds(b, CH)], didx)
                pltpu.sync_copy(norm.at[pl.ds(b, CH)], nb)
                pltpu.sync_copy(xc.at[sidx], rows)
                def row(r, _):
                    v16 = plsc.load_gather(nb, [jnp.full((L,), r, jnp.int32)])
                    for j in range(F // L):
                        rows[r, pl.ds(j * L, L)] = rows[r, pl.ds(j * L, L)] * v16
                    return 0
                lax.fori_loop(0, CH, row, 0)
                pltpu.sync_copy(rows, acc.at[didx], add=True)
                return 0
            lax.fori_loop(0, EPT // CH, chunk, 0)
            plsc.subcore_barrier()

            pltpu.sync_copy(acc.at[pl.ds(s * NPT, NPT)],
                            outc.at[pl.ds(s * NPT, NPT)])

        @pl.when(c == 0)
        def _():
            body(x0, out0)

        @pl.when(c == 1)
        def _():
            body(x1, out1)

    return agg1_kernel


def _make_agg2_kernel(E_pad, N_pad):
    NPT = N_pad // NS
    EPT = E_pad // NS

    @functools.partial(
        pl.kernel,
        out_type=jax.ShapeDtypeStruct((2, N_pad), jnp.float32),
        mesh=plsc.VectorSubcoreMesh(**_MESH),
        compiler_params=pltpu.CompilerParams(needs_layout_passes=False),
        scratch_types=[
            pltpu.VMEM_SHARED((N_pad,), jnp.float32),
            pltpu.VMEM_SHARED((N_pad,), jnp.float32),
            pltpu.VMEM((N_pad // NS,), jnp.float32),
            pltpu.VMEM((CH,), jnp.int32),
            pltpu.VMEM((CH,), jnp.int32),
            pltpu.VMEM((CH,), jnp.float32),
            pltpu.VMEM((CH,), jnp.float32),
            pltpu.VMEM((CH,), jnp.float32),
        ],
    )
    def agg2_kernel(src, dst, norm, h0, h1, pout,
                    acc0, acc1, zb, sidx, didx, nb, g0, g1):
        c = lax.axis_index("c")
        s = lax.axis_index("s")
        z16 = jnp.zeros((L,), jnp.float32)

        @pl.when(c == 0)
        def _():
            def zero_body(k, _):
                zb[pl.ds(k * L, L)] = z16
                return 0
            lax.fori_loop(0, NPT // L, zero_body, 0)
            pltpu.sync_copy(zb, acc0.at[pl.ds(s * NPT, NPT)])
            pltpu.sync_copy(zb, acc1.at[pl.ds(s * NPT, NPT)])
            plsc.subcore_barrier()

            def chunk(k, _):
                b = s * EPT + k * CH
                pltpu.sync_copy(src.at[pl.ds(b, CH)], sidx)
                pltpu.sync_copy(dst.at[pl.ds(b, CH)], didx)
                pltpu.sync_copy(norm.at[pl.ds(b, CH)], nb)
                pltpu.sync_copy(h0.at[sidx], g0)
                pltpu.sync_copy(h1.at[sidx], g1)
                for j in range(CH // L):
                    sl = pl.ds(j * L, L)
                    n16 = nb[sl]
                    g0[sl] = g0[sl] * n16
                    g1[sl] = g1[sl] * n16
                pltpu.sync_copy(g0, acc0.at[didx], add=True)
                pltpu.sync_copy(g1, acc1.at[didx], add=True)
                return 0
            lax.fori_loop(0, EPT // CH, chunk, 0)
            plsc.subcore_barrier()

            pltpu.sync_copy(acc0.at[pl.ds(s * NPT, NPT)],
                            pout.at[0, pl.ds(s * NPT, NPT)])
            pltpu.sync_copy(acc1.at[pl.ds(s * NPT, NPT)],
                            pout.at[1, pl.ds(s * NPT, NPT)])

    return agg2_kernel


def _make_mlp_kernel(N_pad, D, H, OUT):
    BR = 256
    F = D // 2

    def body(a0, a1, w1, b1r, w2, o):
        w1v = w1[...]
        h = (jnp.dot(a0[...], w1v[:F, :], preferred_element_type=jnp.float32)
             + jnp.dot(a1[...], w1v[F:, :], preferred_element_type=jnp.float32)
             + b1r[...])
        h = jnp.maximum(h, 0.0)
        o[...] = jnp.dot(h, w2[...], preferred_element_type=jnp.float32)

    return pl.pallas_call(
        body,
        grid=(N_pad // BR,),
        in_specs=[
            pl.BlockSpec((BR, F), lambda i: (i, 0)),
            pl.BlockSpec((BR, F), lambda i: (i, 0)),
            pl.BlockSpec((D, H), lambda i: (0, 0)),
            pl.BlockSpec((1, H), lambda i: (0, 0)),
            pl.BlockSpec((H, OUT), lambda i: (0, 0)),
        ],
        out_specs=pl.BlockSpec((BR, OUT), lambda i: (i, 0)),
        out_shape=jax.ShapeDtypeStruct((N_pad, OUT), jnp.float32),
    )


def kernel(x, edge_index, edge_weight, W1, b1, W2, b2):
    N, D = x.shape
    E = edge_index.shape[1]
    H = W1.shape[1]
    OUT = W2.shape[1]

    N_pad = -(-N // (NS * L)) * (NS * L)
    E1 = E + N
    step = NS * NC * CH
    E_pad = -(-E1 // step) * step

    src = edge_index[0].astype(jnp.int32)
    dst = edge_index[1].astype(jnp.int32)
    loop = jnp.arange(N, dtype=jnp.int32)
    padz = jnp.zeros((E_pad - E1,), jnp.int32)
    srcp = jnp.concatenate([src, loop, padz])
    dstp = jnp.concatenate([dst, loop, padz])
    ewp = jnp.concatenate([edge_weight.astype(jnp.float32),
                           jnp.ones((N,), jnp.float32),
                           jnp.zeros((E_pad - E1,), jnp.float32)])

    xp = jnp.pad(x.astype(jnp.float32), ((0, N_pad - N), (0, 0)))
    x0 = xp[:, : D // 2]
    x1 = xp[:, D // 2:]

    norm = _make_norm_kernel(E_pad, N_pad)(srcp, dstp, ewp)
    agg0, agg1 = _make_agg1_kernel(E_pad, N_pad, D // 2)(srcp, dstp, norm, x0, x1)
    hw2 = _make_mlp_kernel(N_pad, D, H, OUT)(agg0, agg1, W1, b1.reshape(1, H), W2)
    h0 = hw2[:, 0]
    h1 = hw2[:, 1]
    pout = _make_agg2_kernel(E_pad, N_pad)(srcp, dstp, norm, h0, h1)
    out = jnp.stack([pout[0, :N], pout[1, :N]], axis=1) + b2
    return out


# speedup vs baseline: 12.9056x; 2.3637x over previous
import functools

import jax
import jax.numpy as jnp
from jax import lax
from jax.experimental import pallas as pl
from jax.experimental.pallas import tpu as pltpu
from jax.experimental.pallas import tpu_sc as plsc

NC = 2
NS = 16
L = 16
CH = 112

_MESH = dict(core_axis_name="c", subcore_axis_name="s")
_NOLAYOUT = dict(compiler_params=pltpu.CompilerParams(needs_layout_passes=False))


def _rsqrt16(x):
    i = lax.bitcast_convert_type(x, jnp.int32)
    i = jnp.int32(0x5F3759DF) - (i >> 1)
    y = lax.bitcast_convert_type(i, jnp.float32)
    for _ in range(3):
        y = y * (1.5 - 0.5 * x * y * y)
    return y


def _full16(v):
    return jnp.full((L,), v, jnp.int32)


def _make_norm_kernel(E_pad, N_pad):
    NPT = N_pad // NS
    NCH = E_pad // NS // CH

    @functools.partial(
        pl.kernel,
        out_type=jax.ShapeDtypeStruct((E_pad // CH, 1, CH), jnp.float32),
        mesh=plsc.VectorSubcoreMesh(**_MESH),
        **_NOLAYOUT,
        scratch_types=[
            pltpu.VMEM_SHARED((N_pad,), jnp.float32),
            pltpu.VMEM_SHARED((N_pad,), jnp.float32),
            pltpu.VMEM((N_pad,), jnp.float32),
            pltpu.VMEM((NPT,), jnp.float32),
            pltpu.VMEM((NCH, 1, CH), jnp.int32),
            pltpu.VMEM((NCH, 1, CH), jnp.int32),
            pltpu.VMEM((NCH, 1, CH), jnp.float32),
            pltpu.VMEM((NCH, 1, CH), jnp.float32),
            pltpu.SemaphoreType.DMA,
        ],
    )
    def norm_kernel(src3, dst3, ew3, norm_out,
                    deg_sp, dinv_sp, dinv_loc, slcb, sl, dl, el, nl, sem):
        s = lax.axis_index("s")
        z16 = jnp.zeros((L,), jnp.float32)
        row0 = s * NCH

        pltpu.sync_copy(src3.at[pl.ds(row0, NCH)], sl)
        pltpu.sync_copy(dst3.at[pl.ds(row0, NCH)], dl)
        pltpu.sync_copy(ew3.at[pl.ds(row0, NCH)], el)

        def zero_body(k, _):
            slcb[pl.ds(k * L, L)] = z16
            return 0
        lax.fori_loop(0, NPT // L, zero_body, 0)
        pltpu.sync_copy(slcb, deg_sp.at[pl.ds(s * NPT, NPT)])
        plsc.subcore_barrier()

        GF = 12
        for g in range(0, NCH, GF):
            n = min(GF, NCH - g)
            for t in range(n):
                pltpu.async_copy(el.at[g + t, 0], deg_sp.at[dl.at[g + t, 0]],
                                 sem, add=True)
            for t in range(n):
                pltpu.make_async_copy(el.at[g + t, 0],
                                      deg_sp.at[dl.at[g + t, 0]], sem).wait()
        plsc.subcore_barrier()

        pltpu.sync_copy(deg_sp.at[pl.ds(s * NPT, NPT)], slcb)
        def rsq_body(k, _):
            v = slcb[pl.ds(k * L, L)]
            slcb[pl.ds(k * L, L)] = _rsqrt16(v)
            return 0
        lax.fori_loop(0, NPT // L, rsq_body, 0)
        pltpu.sync_copy(slcb, dinv_sp.at[pl.ds(s * NPT, NPT)])
        plsc.subcore_barrier()

        pltpu.sync_copy(dinv_sp, dinv_loc)

        def nrm_body(k, _):
            for j in range(CH // L):
                slj = pl.ds(j * L, L)
                a = plsc.load_gather(dinv_loc, [sl[k, 0, slj]])
                b = plsc.load_gather(dinv_loc, [dl[k, 0, slj]])
                nl[k, 0, slj] = a * el[k, 0, slj] * b
            return 0
        lax.fori_loop(0, NCH, nrm_body, 0)
        pltpu.sync_copy(nl, norm_out.at[pl.ds(row0, NCH)])

    return norm_kernel


def _make_agg1_kernel(E_pad, N_pad, F):
    NPT = N_pad // NS
    NCH = E_pad // NS // CH
    NB = 3

    @functools.partial(
        pl.kernel,
        out_type=(jax.ShapeDtypeStruct((N_pad, F), jnp.float32),
                  jax.ShapeDtypeStruct((N_pad, F), jnp.float32)),
        mesh=plsc.VectorSubcoreMesh(**_MESH),
        **_NOLAYOUT,
        scratch_types=[
            pltpu.VMEM_SHARED((N_pad, F), jnp.float32),
            pltpu.VMEM((CH, F), jnp.float32),
            pltpu.VMEM((CH, F), jnp.float32),
            pltpu.VMEM((CH, F), jnp.float32),
            pltpu.VMEM((CH,), jnp.int32),
            pltpu.VMEM((CH,), jnp.int32),
            pltpu.VMEM((CH,), jnp.int32),
            pltpu.VMEM((CH,), jnp.int32),
            pltpu.VMEM((CH,), jnp.int32),
            pltpu.VMEM((CH,), jnp.int32),
            pltpu.VMEM((CH,), jnp.float32),
            pltpu.VMEM((CH,), jnp.float32),
            pltpu.VMEM((CH,), jnp.float32),
            pltpu.SemaphoreType.DMA,
            pltpu.SemaphoreType.DMA,
            pltpu.SemaphoreType.DMA,
            pltpu.SemaphoreType.DMA,
            pltpu.SemaphoreType.DMA,
            pltpu.SemaphoreType.DMA,
            pltpu.SemaphoreType.DMA,
            pltpu.SemaphoreType.DMA,
            pltpu.SemaphoreType.DMA,
        ],
    )
    def agg1_kernel(src3, dst3, norm3, x0, x1, out0, out1,
                    acc, r0, r1, r2, sa0, sa1, sa2, da0, da1, da2,
                    na0, na1, na2, g0, g1, g2, s0, s1, s2, i0, i1, i2):
        c = lax.axis_index("c")
        s = lax.axis_index("s")
        rows = (r0, r1, r2)
        sidx = (sa0, sa1, sa2)
        didx = (da0, da1, da2)
        nbuf = (na0, na1, na2)
        gsem = (g0, g1, g2)
        ssem = (s0, s1, s2)
        isem = (i0, i1, i2)
        z16 = jnp.zeros((L,), jnp.float32)
        row0 = s * NCH

        def idx_fetch(k, j):
            pltpu.async_copy(src3.at[row0 + k, 0], sidx[j], isem[j])
            pltpu.async_copy(dst3.at[row0 + k, 0], didx[j], isem[j])
            pltpu.async_copy(norm3.at[row0 + k, 0], nbuf[j], isem[j])

        def idx_wait(j):
            pltpu.make_async_copy(src3.at[row0, 0], sidx[j], isem[j]).wait()
            pltpu.make_async_copy(dst3.at[row0, 0], didx[j], isem[j]).wait()
            pltpu.make_async_copy(norm3.at[row0, 0], nbuf[j], isem[j]).wait()

        def body(xc, outc):
            for rr in range(L):
                for j in range(F // L):
                    r0[rr, pl.ds(j * L, L)] = z16
            def zcp(k, _):
                pltpu.sync_copy(r0.at[pl.ds(0, L)],
                                acc.at[pl.ds(s * NPT + k * L, L)])
                return 0
            lax.fori_loop(0, NPT // L, zcp, 0)
            plsc.subcore_barrier()

            def scale(buf, nb):
                def rowf(r, _):
                    v16 = plsc.load_gather(nb, [_full16(r)])
                    for j in range(F // L):
                        buf[r, pl.ds(j * L, L)] = buf[r, pl.ds(j * L, L)] * v16
                    return 0
                lax.fori_loop(0, CH, rowf, 0, unroll=4)

            idx_fetch(0, 0)
            idx_fetch(1, 1)
            idx_wait(0)
            pltpu.async_copy(xc.at[sidx[0]], rows[0], gsem[0])

            def outer(i, _):
                for b in range(NB):
                    k = i * NB + b
                    bn1 = (b + 1) % NB
                    bn2 = (b + 2) % NB
                    pltpu.make_async_copy(xc.at[sidx[b]], rows[b],
                                          gsem[b]).wait()
                    scale(rows[b], nbuf[b])
                    pltpu.async_copy(rows[b], acc.at[didx[b]], ssem[b],
                                     add=True)
                    @pl.when(k >= 1)
                    def _():
                        pltpu.make_async_copy(rows[bn2], acc.at[didx[bn2]],
                                              ssem[bn2]).wait()
                    @pl.when(k + 2 < NCH)
                    def _():
                        idx_fetch(k + 2, bn2)
                    @pl.when(k + 1 < NCH)
                    def _():
                        idx_wait(bn1)
                        pltpu.async_copy(xc.at[sidx[bn1]], rows[bn1],
                                         gsem[bn1])
                return 0
            lax.fori_loop(0, NCH // NB, outer, 0)
            pltpu.make_async_copy(rows[(NCH - 1) % NB],
                                  acc.at[didx[(NCH - 1) % NB]],
                                  ssem[(NCH - 1) % NB]).wait()
            plsc.subcore_barrier()

            pltpu.sync_copy(acc.at[pl.ds(s * NPT, NPT)],
                            outc.at[pl.ds(s * NPT, NPT)])

        @pl.when(c == 0)
        def _():
            body(x0, out0)

        @pl.when(c == 1)
        def _():
            body(x1, out1)

    return agg1_kernel


def _make_agg2_kernel(E_pad, N_pad):
    NPT = N_pad // NS
    NCH = E_pad // (NS * NC) // CH

    @functools.partial(
        pl.kernel,
        out_type=(jax.ShapeDtypeStruct((2, N_pad), jnp.float32),
                  jax.ShapeDtypeStruct((2, N_pad), jnp.float32)),
        mesh=plsc.VectorSubcoreMesh(**_MESH),
        **_NOLAYOUT,
        scratch_types=[
            pltpu.VMEM_SHARED((N_pad,), jnp.float32),
            pltpu.VMEM_SHARED((N_pad,), jnp.float32),
            pltpu.VMEM((N_pad,), jnp.float32),
            pltpu.VMEM((N_pad,), jnp.float32),
            pltpu.VMEM((NPT,), jnp.float32),
            pltpu.VMEM((NCH, 1, CH), jnp.int32),
            pltpu.VMEM((NCH, 1, CH), jnp.int32),
            pltpu.VMEM((NCH, 1, CH), jnp.float32),
            pltpu.VMEM((NCH, 1, CH), jnp.float32),
            pltpu.VMEM((NCH, 1, CH), jnp.float32),
            pltpu.SemaphoreType.DMA,
        ],
    )
    def agg2_kernel(src3, dst3, norm3, h0, h1, pA, pB,
                    acc0, acc1, h0l, h1l, slcb, sl, dl, nl, q0, q1, sem):
        c = lax.axis_index("c")
        s = lax.axis_index("s")
        w = c * NS + s
        row0 = w * NCH
        z16 = jnp.zeros((L,), jnp.float32)

        pltpu.sync_copy(h0, h0l)
        pltpu.sync_copy(h1, h1l)
        pltpu.sync_copy(src3.at[pl.ds(row0, NCH)], sl)
        pltpu.sync_copy(dst3.at[pl.ds(row0, NCH)], dl)
        pltpu.sync_copy(norm3.at[pl.ds(row0, NCH)], nl)

        def zero_body(k, _):
            slcb[pl.ds(k * L, L)] = z16
            return 0
        lax.fori_loop(0, NPT // L, zero_body, 0)
        pltpu.sync_copy(slcb, acc0.at[pl.ds(s * NPT, NPT)])
        pltpu.sync_copy(slcb, acc1.at[pl.ds(s * NPT, NPT)])
        plsc.subcore_barrier()

        def chunk(k, _):
            for j in range(CH // L):
                slj = pl.ds(j * L, L)
                s16 = sl[k, 0, slj]
                n16 = nl[k, 0, slj]
                q0[k, 0, slj] = plsc.load_gather(h0l, [s16]) * n16
                q1[k, 0, slj] = plsc.load_gather(h1l, [s16]) * n16
            return 0
        lax.fori_loop(0, NCH, chunk, 0)

        GF = 6
        for g in range(0, NCH, GF):
            n = min(GF, NCH - g)
            for t in range(n):
                pltpu.async_copy(q0.at[g + t, 0], acc0.at[dl.at[g + t, 0]],
                                 sem, add=True)
                pltpu.async_copy(q1.at[g + t, 0], acc1.at[dl.at[g + t, 0]],
                                 sem, add=True)
            for t in range(n):
                pltpu.make_async_copy(q0.at[g + t, 0],
                                      acc0.at[dl.at[g + t, 0]], sem).wait()
                pltpu.make_async_copy(q1.at[g + t, 0],
                                      acc1.at[dl.at[g + t, 0]], sem).wait()
        plsc.subcore_barrier()

        @pl.when(c == 0)
        def _():
            pltpu.sync_copy(acc0.at[pl.ds(s * NPT, NPT)],
                            pA.at[0, pl.ds(s * NPT, NPT)])
            pltpu.sync_copy(acc1.at[pl.ds(s * NPT, NPT)],
                            pA.at[1, pl.ds(s * NPT, NPT)])

        @pl.when(c == 1)
        def _():
            pltpu.sync_copy(acc0.at[pl.ds(s * NPT, NPT)],
                            pB.at[0, pl.ds(s * NPT, NPT)])
            pltpu.sync_copy(acc1.at[pl.ds(s * NPT, NPT)],
                            pB.at[1, pl.ds(s * NPT, NPT)])

    return agg2_kernel


def _make_mlp_kernel(N_pad, D, H, OUT):
    BR = 256
    F = D // 2

    def body(a0, a1, w1, b1r, w2, o):
        w1v = w1[...]
        h = (jnp.dot(a0[...], w1v[:F, :], preferred_element_type=jnp.float32)
             + jnp.dot(a1[...], w1v[F:, :], preferred_element_type=jnp.float32)
             + b1r[...])
        h = jnp.maximum(h, 0.0)
        o[...] = jnp.dot(h, w2[...], preferred_element_type=jnp.float32)

    return pl.pallas_call(
        body,
        grid=(N_pad // BR,),
        in_specs=[
            pl.BlockSpec((BR, F), lambda i: (i, 0)),
            pl.BlockSpec((BR, F), lambda i: (i, 0)),
            pl.BlockSpec((D, H), lambda i: (0, 0)),
            pl.BlockSpec((1, H), lambda i: (0, 0)),
            pl.BlockSpec((H, OUT), lambda i: (0, 0)),
        ],
        out_specs=pl.BlockSpec((BR, OUT), lambda i: (i, 0)),
        out_shape=jax.ShapeDtypeStruct((N_pad, OUT), jnp.float32),
    )


def kernel(x, edge_index, edge_weight, W1, b1, W2, b2):
    N, D = x.shape
    E = edge_index.shape[1]
    H = W1.shape[1]
    OUT = W2.shape[1]

    N_pad = -(-N // (NS * L)) * (NS * L)
    E1 = E + N
    step = NS * NC * CH * 3
    E_pad = -(-E1 // step) * step

    src = edge_index[0].astype(jnp.int32)
    dst = edge_index[1].astype(jnp.int32)
    loop = jnp.arange(N, dtype=jnp.int32)
    padz = jnp.zeros((E_pad - E1,), jnp.int32)
    srcp = jnp.concatenate([src, loop, padz]).reshape(E_pad // CH, 1, CH)
    dstp = jnp.concatenate([dst, loop, padz]).reshape(E_pad // CH, 1, CH)
    ewp = jnp.concatenate([edge_weight.astype(jnp.float32),
                           jnp.ones((N,), jnp.float32),
                           jnp.zeros((E_pad - E1,), jnp.float32)]
                          ).reshape(E_pad // CH, 1, CH)

    xp = jnp.pad(x.astype(jnp.float32), ((0, N_pad - N), (0, 0)))
    x0 = xp[:, : D // 2]
    x1 = xp[:, D // 2:]

    norm = _make_norm_kernel(E_pad, N_pad)(srcp, dstp, ewp)
    agg0, agg1 = _make_agg1_kernel(E_pad, N_pad, D // 2)(srcp, dstp, norm,
                                                         x0, x1)
    hw2 = _make_mlp_kernel(N_pad, D, H, OUT)(agg0, agg1, W1,
                                             b1.reshape(1, H), W2)
    h0 = hw2[:, 0]
    h1 = hw2[:, 1]
    pA, pB = _make_agg2_kernel(E_pad, N_pad)(srcp, dstp, norm, h0, h1)
    q = pA + pB
    out = q[:, :N].T + b2
    return out


# speedup vs baseline: 15.8812x; 1.2306x over previous
import functools

import jax
import jax.numpy as jnp
from jax import lax
from jax.experimental import pallas as pl
from jax.experimental.pallas import tpu as pltpu
from jax.experimental.pallas import tpu_sc as plsc

NC = 2
NS = 16
L = 16
CH = 112

_MESH = dict(core_axis_name="c", subcore_axis_name="s")
_NOLAYOUT = dict(compiler_params=pltpu.CompilerParams(needs_layout_passes=False))


def _rsqrt16(x):
    i = lax.bitcast_convert_type(x, jnp.int32)
    i = jnp.int32(0x5F3759DF) - (i >> 1)
    y = lax.bitcast_convert_type(i, jnp.float32)
    for _ in range(3):
        y = y * (1.5 - 0.5 * x * y * y)
    return y


def _full16(v):
    return jnp.full((L,), v, jnp.int32)


def _make_norm_kernel(E_pad, N_pad):
    NPT = N_pad // NS
    NCH = E_pad // NS // CH

    @functools.partial(
        pl.kernel,
        out_type=jax.ShapeDtypeStruct((E_pad // CH, 1, CH), jnp.float32),
        mesh=plsc.VectorSubcoreMesh(**_MESH),
        **_NOLAYOUT,
        scratch_types=[
            pltpu.VMEM_SHARED((N_pad,), jnp.float32),
            pltpu.VMEM_SHARED((N_pad,), jnp.float32),
            pltpu.VMEM((N_pad,), jnp.float32),
            pltpu.VMEM((NPT,), jnp.float32),
            pltpu.VMEM((NCH, 1, CH), jnp.int32),
            pltpu.VMEM((NCH, 1, CH), jnp.int32),
            pltpu.VMEM((NCH, 1, CH), jnp.float32),
            pltpu.VMEM((NCH, 1, CH), jnp.float32),
            pltpu.SemaphoreType.DMA,
        ],
    )
    def norm_kernel(src3, dst3, ew3, norm_out,
                    deg_sp, dinv_sp, dinv_loc, slcb, sl, dl, el, nl, sem):
        s = lax.axis_index("s")
        z16 = jnp.zeros((L,), jnp.float32)
        row0 = s * NCH

        pltpu.sync_copy(src3.at[pl.ds(row0, NCH)], sl)
        pltpu.sync_copy(dst3.at[pl.ds(row0, NCH)], dl)
        pltpu.sync_copy(ew3.at[pl.ds(row0, NCH)], el)

        def zero_body(k, _):
            slcb[pl.ds(k * L, L)] = z16
            return 0
        lax.fori_loop(0, NPT // L, zero_body, 0)
        pltpu.sync_copy(slcb, deg_sp.at[pl.ds(s * NPT, NPT)])
        plsc.subcore_barrier()

        GF = 12
        for g in range(0, NCH, GF):
            n = min(GF, NCH - g)
            for t in range(n):
                pltpu.async_copy(el.at[g + t, 0], deg_sp.at[dl.at[g + t, 0]],
                                 sem, add=True)
            for t in range(n):
                pltpu.make_async_copy(el.at[g + t, 0],
                                      deg_sp.at[dl.at[g + t, 0]], sem).wait()
        plsc.subcore_barrier()

        pltpu.sync_copy(deg_sp.at[pl.ds(s * NPT, NPT)], slcb)
        def rsq_body(k, _):
            v = slcb[pl.ds(k * L, L)]
            slcb[pl.ds(k * L, L)] = _rsqrt16(v)
            return 0
        lax.fori_loop(0, NPT // L, rsq_body, 0)
        pltpu.sync_copy(slcb, dinv_sp.at[pl.ds(s * NPT, NPT)])
        plsc.subcore_barrier()

        pltpu.sync_copy(dinv_sp, dinv_loc)

        def nrm_body(k, _):
            for j in range(CH // L):
                slj = pl.ds(j * L, L)
                a = plsc.load_gather(dinv_loc, [sl[k, 0, slj]])
                b = plsc.load_gather(dinv_loc, [dl[k, 0, slj]])
                nl[k, 0, slj] = a * el[k, 0, slj] * b
            return 0
        lax.fori_loop(0, NCH, nrm_body, 0)
        pltpu.sync_copy(nl, norm_out.at[pl.ds(row0, NCH)])

    return norm_kernel


def _make_agg1_kernel(E_pad, N_pad, F):
    NPT = N_pad // NS
    NCH = E_pad // NS // CH
    NB = 3

    @functools.partial(
        pl.kernel,
        out_type=(jax.ShapeDtypeStruct((N_pad, F), jnp.float32),
                  jax.ShapeDtypeStruct((N_pad, F), jnp.float32)),
        mesh=plsc.VectorSubcoreMesh(**_MESH),
        **_NOLAYOUT,
        scratch_types=[
            pltpu.VMEM_SHARED((N_pad, F), jnp.float32),
            pltpu.VMEM((CH, F), jnp.float32),
            pltpu.VMEM((CH, F), jnp.float32),
            pltpu.VMEM((CH, F), jnp.float32),
            pltpu.VMEM((CH,), jnp.int32),
            pltpu.VMEM((CH,), jnp.int32),
            pltpu.VMEM((CH,), jnp.int32),
            pltpu.VMEM((CH,), jnp.int32),
            pltpu.VMEM((CH,), jnp.int32),
            pltpu.VMEM((CH,), jnp.int32),
            pltpu.VMEM((CH,), jnp.int32),
            pltpu.VMEM((CH,), jnp.int32),
            pltpu.VMEM((CH,), jnp.float32),
            pltpu.VMEM((CH,), jnp.float32),
            pltpu.VMEM((CH,), jnp.float32),
            pltpu.VMEM((CH,), jnp.float32),
            pltpu.SemaphoreType.DMA,
            pltpu.SemaphoreType.DMA,
            pltpu.SemaphoreType.DMA,
            pltpu.SemaphoreType.DMA,
            pltpu.SemaphoreType.DMA,
            pltpu.SemaphoreType.DMA,
            pltpu.SemaphoreType.DMA,
            pltpu.SemaphoreType.DMA,
            pltpu.SemaphoreType.DMA,
            pltpu.SemaphoreType.DMA,
        ],
    )
    def agg1_kernel(src3, dst3, norm3, x0, x1, out0, out1,
                    acc, r0, r1, r2, sa0, sa1, sa2, sa3, da0, da1, da2, da3,
                    na0, na1, na2, na3, g0, g1, g2, s0, s1, s2,
                    i0, i1, i2, i3):
        c = lax.axis_index("c")
        s = lax.axis_index("s")
        rows = (r0, r1, r2)
        sidx = (sa0, sa1, sa2, sa3)
        didx = (da0, da1, da2, da3)
        nbuf = (na0, na1, na2, na3)
        gsem = (g0, g1, g2)
        ssem = (s0, s1, s2)
        isem = (i0, i1, i2, i3)
        z16 = jnp.zeros((L,), jnp.float32)
        row0 = s * NCH
        LCM = 12

        def idx_fetch(k, j):
            pltpu.async_copy(src3.at[row0 + k, 0], sidx[j], isem[j])
            pltpu.async_copy(dst3.at[row0 + k, 0], didx[j], isem[j])
            pltpu.async_copy(norm3.at[row0 + k, 0], nbuf[j], isem[j])

        def idx_wait(j):
            pltpu.make_async_copy(src3.at[row0, 0], sidx[j], isem[j]).wait()
            pltpu.make_async_copy(dst3.at[row0, 0], didx[j], isem[j]).wait()
            pltpu.make_async_copy(norm3.at[row0, 0], nbuf[j], isem[j]).wait()

        def body(xc, outc):
            for rr in range(L):
                for j in range(F // L):
                    r0[rr, pl.ds(j * L, L)] = z16
            def zcp(k, _):
                pltpu.sync_copy(r0.at[pl.ds(0, L)],
                                acc.at[pl.ds(s * NPT + k * L, L)])
                return 0
            lax.fori_loop(0, NPT // L, zcp, 0)
            plsc.subcore_barrier()

            def scale(buf, nb):
                def rowf(r, _):
                    v16 = plsc.load_gather(nb, [_full16(r)])
                    for j in range(F // L):
                        buf[r, pl.ds(j * L, L)] = buf[r, pl.ds(j * L, L)] * v16
                    return 0
                lax.fori_loop(0, CH, rowf, 0, unroll=4)

            idx_fetch(0, 0)
            idx_fetch(1, 1)
            idx_wait(0)
            pltpu.async_copy(xc.at[sidx[0]], rows[0], gsem[0])

            def outer(i, _):
                for kk in range(LCM):
                    k = i * LCM + kk
                    j3 = kk % 3
                    j4 = kk % 4
                    pltpu.make_async_copy(xc.at[sidx[j4]], rows[j3],
                                          gsem[j3]).wait()
                    @pl.when(k >= 2)
                    def _():
                        pltpu.make_async_copy(rows[(j3 + 1) % 3],
                                              acc.at[didx[(j4 + 2) % 4]],
                                              ssem[(j3 + 1) % 3]).wait()
                    @pl.when(k + 1 < NCH)
                    def _():
                        idx_wait((j4 + 1) % 4)
                        pltpu.async_copy(xc.at[sidx[(j4 + 1) % 4]],
                                         rows[(j3 + 1) % 3],
                                         gsem[(j3 + 1) % 3])
                    @pl.when(k + 2 < NCH)
                    def _():
                        idx_fetch(k + 2, (j4 + 2) % 4)
                    scale(rows[j3], nbuf[j4])
                    pltpu.async_copy(rows[j3], acc.at[didx[j4]], ssem[j3],
                                     add=True)
                return 0
            lax.fori_loop(0, NCH // LCM, outer, 0)
            for k in (NCH - 2, NCH - 1):
                pltpu.make_async_copy(rows[k % 3], acc.at[didx[k % 4]],
                                      ssem[k % 3]).wait()
            plsc.subcore_barrier()

            pltpu.sync_copy(acc.at[pl.ds(s * NPT, NPT)],
                            outc.at[pl.ds(s * NPT, NPT)])

        @pl.when(c == 0)
        def _():
            body(x0, out0)

        @pl.when(c == 1)
        def _():
            body(x1, out1)

    return agg1_kernel


def _make_agg2_kernel(E_pad, N_pad):
    NPT = N_pad // NS
    NCH = E_pad // (NS * NC) // CH

    @functools.partial(
        pl.kernel,
        out_type=(jax.ShapeDtypeStruct((2, N_pad), jnp.float32),
                  jax.ShapeDtypeStruct((2, N_pad), jnp.float32)),
        mesh=plsc.VectorSubcoreMesh(**_MESH),
        **_NOLAYOUT,
        scratch_types=[
            pltpu.VMEM_SHARED((N_pad,), jnp.float32),
            pltpu.VMEM_SHARED((N_pad,), jnp.float32),
            pltpu.VMEM((N_pad,), jnp.float32),
            pltpu.VMEM((N_pad,), jnp.float32),
            pltpu.VMEM((NPT,), jnp.float32),
            pltpu.VMEM((NCH, 1, CH), jnp.int32),
            pltpu.VMEM((NCH, 1, CH), jnp.int32),
            pltpu.VMEM((NCH, 1, CH), jnp.float32),
            pltpu.VMEM((NCH, 1, CH), jnp.float32),
            pltpu.VMEM((NCH, 1, CH), jnp.float32),
            pltpu.SemaphoreType.DMA,
        ],
    )
    def agg2_kernel(src3, dst3, norm3, h0, h1, pA, pB,
                    acc0, acc1, h0l, h1l, slcb, sl, dl, nl, q0, q1, sem):
        c = lax.axis_index("c")
        s = lax.axis_index("s")
        w = c * NS + s
        row0 = w * NCH
        z16 = jnp.zeros((L,), jnp.float32)

        pltpu.sync_copy(h0, h0l)
        pltpu.sync_copy(h1, h1l)
        pltpu.sync_copy(src3.at[pl.ds(row0, NCH)], sl)
        pltpu.sync_copy(dst3.at[pl.ds(row0, NCH)], dl)
        pltpu.sync_copy(norm3.at[pl.ds(row0, NCH)], nl)

        def zero_body(k, _):
            slcb[pl.ds(k * L, L)] = z16
            return 0
        lax.fori_loop(0, NPT // L, zero_body, 0)
        pltpu.sync_copy(slcb, acc0.at[pl.ds(s * NPT, NPT)])
        pltpu.sync_copy(slcb, acc1.at[pl.ds(s * NPT, NPT)])
        plsc.subcore_barrier()

        def chunk(k, _):
            for j in range(CH // L):
                slj = pl.ds(j * L, L)
                s16 = sl[k, 0, slj]
                n16 = nl[k, 0, slj]
                q0[k, 0, slj] = plsc.load_gather(h0l, [s16]) * n16
                q1[k, 0, slj] = plsc.load_gather(h1l, [s16]) * n16
            return 0
        lax.fori_loop(0, NCH, chunk, 0)

        GF = 6
        for g in range(0, NCH, GF):
            n = min(GF, NCH - g)
            for t in range(n):
                pltpu.async_copy(q0.at[g + t, 0], acc0.at[dl.at[g + t, 0]],
                                 sem, add=True)
                pltpu.async_copy(q1.at[g + t, 0], acc1.at[dl.at[g + t, 0]],
                                 sem, add=True)
            for t in range(n):
                pltpu.make_async_copy(q0.at[g + t, 0],
                                      acc0.at[dl.at[g + t, 0]], sem).wait()
                pltpu.make_async_copy(q1.at[g + t, 0],
                                      acc1.at[dl.at[g + t, 0]], sem).wait()
        plsc.subcore_barrier()

        @pl.when(c == 0)
        def _():
            pltpu.sync_copy(acc0.at[pl.ds(s * NPT, NPT)],
                            pA.at[0, pl.ds(s * NPT, NPT)])
            pltpu.sync_copy(acc1.at[pl.ds(s * NPT, NPT)],
                            pA.at[1, pl.ds(s * NPT, NPT)])

        @pl.when(c == 1)
        def _():
            pltpu.sync_copy(acc0.at[pl.ds(s * NPT, NPT)],
                            pB.at[0, pl.ds(s * NPT, NPT)])
            pltpu.sync_copy(acc1.at[pl.ds(s * NPT, NPT)],
                            pB.at[1, pl.ds(s * NPT, NPT)])

    return agg2_kernel


def _make_mlp_kernel(N_pad, D, H, OUT):
    BR = 256
    F = D // 2

    def body(a0, a1, w1, b1r, w2, o):
        w1v = w1[...]
        h = (jnp.dot(a0[...], w1v[:F, :], preferred_element_type=jnp.float32)
             + jnp.dot(a1[...], w1v[F:, :], preferred_element_type=jnp.float32)
             + b1r[...])
        h = jnp.maximum(h, 0.0)
        o[...] = jnp.dot(h, w2[...], preferred_element_type=jnp.float32)

    return pl.pallas_call(
        body,
        grid=(N_pad // BR,),
        in_specs=[
            pl.BlockSpec((BR, F), lambda i: (i, 0)),
            pl.BlockSpec((BR, F), lambda i: (i, 0)),
            pl.BlockSpec((D, H), lambda i: (0, 0)),
            pl.BlockSpec((1, H), lambda i: (0, 0)),
            pl.BlockSpec((H, OUT), lambda i: (0, 0)),
        ],
        out_specs=pl.BlockSpec((BR, OUT), lambda i: (i, 0)),
        out_shape=jax.ShapeDtypeStruct((N_pad, OUT), jnp.float32),
    )


def kernel(x, edge_index, edge_weight, W1, b1, W2, b2):
    N, D = x.shape
    E = edge_index.shape[1]
    H = W1.shape[1]
    OUT = W2.shape[1]

    N_pad = -(-N // (NS * L)) * (NS * L)
    E1 = E + N
    step = NS * NC * CH * 3
    E_pad = -(-E1 // step) * step

    src = edge_index[0].astype(jnp.int32)
    dst = edge_index[1].astype(jnp.int32)
    loop = jnp.arange(N, dtype=jnp.int32)
    padz = jnp.zeros((E_pad - E1,), jnp.int32)
    srcp = jnp.concatenate([src, loop, padz]).reshape(E_pad // CH, 1, CH)
    dstp = jnp.concatenate([dst, loop, padz]).reshape(E_pad // CH, 1, CH)
    ewp = jnp.concatenate([edge_weight.astype(jnp.float32),
                           jnp.ones((N,), jnp.float32),
                           jnp.zeros((E_pad - E1,), jnp.float32)]
                          ).reshape(E_pad // CH, 1, CH)

    xp = jnp.pad(x.astype(jnp.float32), ((0, N_pad - N), (0, 0)))
    x0 = xp[:, : D // 2]
    x1 = xp[:, D // 2:]

    norm = _make_norm_kernel(E_pad, N_pad)(srcp, dstp, ewp)
    agg0, agg1 = _make_agg1_kernel(E_pad, N_pad, D // 2)(srcp, dstp, norm,
                                                         x0, x1)
    hw2 = _make_mlp_kernel(N_pad, D, H, OUT)(agg0, agg1, W1,
                                             b1.reshape(1, H), W2)
    h0 = hw2[:, 0]
    h1 = hw2[:, 1]
    pA, pB = _make_agg2_kernel(E_pad, N_pad)(srcp, dstp, norm, h0, h1)
    q = pA + pB
    out = q[:, :N].T + b2
    return out


# speedup vs baseline: 15.9230x; 1.0026x over previous
import functools

import jax
import jax.numpy as jnp
from jax import lax
from jax.experimental import pallas as pl
from jax.experimental.pallas import tpu as pltpu
from jax.experimental.pallas import tpu_sc as plsc

NC = 2
NS = 16
L = 16
CH = 112

_MESH = dict(core_axis_name="c", subcore_axis_name="s")
_NOLAYOUT = dict(compiler_params=pltpu.CompilerParams(needs_layout_passes=False))


def _rsqrt16(x):
    i = lax.bitcast_convert_type(x, jnp.int32)
    i = jnp.int32(0x5F3759DF) - (i >> 1)
    y = lax.bitcast_convert_type(i, jnp.float32)
    for _ in range(3):
        y = y * (1.5 - 0.5 * x * y * y)
    return y


def _full16(v):
    return jnp.full((L,), v, jnp.int32)


def _make_norm_kernel(E_pad, N_pad):
    NPT = N_pad // NS
    NCH = E_pad // NS // CH

    @functools.partial(
        pl.kernel,
        out_type=jax.ShapeDtypeStruct((E_pad // CH, 1, CH), jnp.float32),
        mesh=plsc.VectorSubcoreMesh(**_MESH),
        **_NOLAYOUT,
        scratch_types=[
            pltpu.VMEM_SHARED((N_pad,), jnp.float32),
            pltpu.VMEM_SHARED((N_pad,), jnp.float32),
            pltpu.VMEM((N_pad,), jnp.float32),
            pltpu.VMEM((NPT,), jnp.float32),
            pltpu.VMEM((NCH, 1, CH), jnp.int32),
            pltpu.VMEM((NCH, 1, CH), jnp.int32),
            pltpu.VMEM((NCH, 1, CH), jnp.float32),
            pltpu.VMEM((NCH, 1, CH), jnp.float32),
            pltpu.SemaphoreType.DMA,
        ],
    )
    def norm_kernel(src3, dst3, ew3, norm_out,
                    deg_sp, dinv_sp, dinv_loc, slcb, sl, dl, el, nl, sem):
        s = lax.axis_index("s")
        z16 = jnp.zeros((L,), jnp.float32)
        row0 = s * NCH

        pltpu.sync_copy(src3.at[pl.ds(row0, NCH)], sl)
        pltpu.sync_copy(dst3.at[pl.ds(row0, NCH)], dl)
        pltpu.sync_copy(ew3.at[pl.ds(row0, NCH)], el)

        def zero_body(k, _):
            slcb[pl.ds(k * L, L)] = z16
            return 0
        lax.fori_loop(0, NPT // L, zero_body, 0)
        pltpu.sync_copy(slcb, deg_sp.at[pl.ds(s * NPT, NPT)])
        plsc.subcore_barrier()

        GF = 12
        for g in range(0, NCH, GF):
            n = min(GF, NCH - g)
            for t in range(n):
                pltpu.async_copy(el.at[g + t, 0], deg_sp.at[dl.at[g + t, 0]],
                                 sem, add=True)
            for t in range(n):
                pltpu.make_async_copy(el.at[g + t, 0],
                                      deg_sp.at[dl.at[g + t, 0]], sem).wait()
        plsc.subcore_barrier()

        pltpu.sync_copy(deg_sp.at[pl.ds(s * NPT, NPT)], slcb)
        def rsq_body(k, _):
            v = slcb[pl.ds(k * L, L)]
            slcb[pl.ds(k * L, L)] = _rsqrt16(v)
            return 0
        lax.fori_loop(0, NPT // L, rsq_body, 0)
        pltpu.sync_copy(slcb, dinv_sp.at[pl.ds(s * NPT, NPT)])
        plsc.subcore_barrier()

        pltpu.sync_copy(dinv_sp, dinv_loc)

        def nrm_body(k, _):
            for j in range(CH // L):
                slj = pl.ds(j * L, L)
                a = plsc.load_gather(dinv_loc, [sl[k, 0, slj]])
                b = plsc.load_gather(dinv_loc, [dl[k, 0, slj]])
                nl[k, 0, slj] = a * el[k, 0, slj] * b
            return 0
        lax.fori_loop(0, NCH, nrm_body, 0)
        pltpu.sync_copy(nl, norm_out.at[pl.ds(row0, NCH)])

    return norm_kernel


def _make_agg1_kernel(E_pad, N_pad, F):
    NPT = N_pad // NS
    NCH = E_pad // NS // CH
    NB = 3

    @functools.partial(
        pl.kernel,
        out_type=(jax.ShapeDtypeStruct((N_pad, F), jnp.float32),
                  jax.ShapeDtypeStruct((N_pad, F), jnp.float32)),
        mesh=plsc.VectorSubcoreMesh(**_MESH),
        **_NOLAYOUT,
        scratch_types=[
            pltpu.VMEM_SHARED((N_pad, F), jnp.float32),
            pltpu.VMEM((CH, F), jnp.float32),
            pltpu.VMEM((CH, F), jnp.float32),
            pltpu.VMEM((CH, F), jnp.float32),
            pltpu.VMEM((CH,), jnp.int32),
            pltpu.VMEM((CH,), jnp.int32),
            pltpu.VMEM((CH,), jnp.int32),
            pltpu.VMEM((CH,), jnp.int32),
            pltpu.VMEM((CH,), jnp.int32),
            pltpu.VMEM((CH,), jnp.int32),
            pltpu.VMEM((CH,), jnp.int32),
            pltpu.VMEM((CH,), jnp.int32),
            pltpu.VMEM((CH,), jnp.float32),
            pltpu.VMEM((CH,), jnp.float32),
            pltpu.VMEM((CH,), jnp.float32),
            pltpu.VMEM((CH,), jnp.float32),
            pltpu.SemaphoreType.DMA,
            pltpu.SemaphoreType.DMA,
            pltpu.SemaphoreType.DMA,
            pltpu.SemaphoreType.DMA,
            pltpu.SemaphoreType.DMA,
            pltpu.SemaphoreType.DMA,
            pltpu.SemaphoreType.DMA,
            pltpu.SemaphoreType.DMA,
            pltpu.SemaphoreType.DMA,
            pltpu.SemaphoreType.DMA,
        ],
    )
    def agg1_kernel(src3, dst3, norm3, x0, x1, out0, out1,
                    acc, r0, r1, r2,
                    sa0, sa1, sa2, sa3, da0, da1, da2, da3,
                    na0, na1, na2, na3, g0, g1, g2, s0, s1, s2,
                    i0, i1, i2, i3):
        c = lax.axis_index("c")
        s = lax.axis_index("s")
        rows = (r0, r1, r2)
        sidx = (sa0, sa1, sa2, sa3)
        didx = (da0, da1, da2, da3)
        nbuf = (na0, na1, na2, na3)
        gsem = (g0, g1, g2)
        ssem = (s0, s1, s2)
        isem = (i0, i1, i2, i3)
        z16 = jnp.zeros((L,), jnp.float32)
        row0 = s * NCH
        LCM = 12

        def idx_fetch(k, j):
            pltpu.async_copy(src3.at[row0 + k, 0], sidx[j], isem[j])
            pltpu.async_copy(dst3.at[row0 + k, 0], didx[j], isem[j])
            pltpu.async_copy(norm3.at[row0 + k, 0], nbuf[j], isem[j])

        def idx_wait(j):
            pltpu.make_async_copy(src3.at[row0, 0], sidx[j], isem[j]).wait()
            pltpu.make_async_copy(dst3.at[row0, 0], didx[j], isem[j]).wait()
            pltpu.make_async_copy(norm3.at[row0, 0], nbuf[j], isem[j]).wait()

        def body(xc, outc):
            for rr in range(L):
                for j in range(F // L):
                    r0[rr, pl.ds(j * L, L)] = z16
            def zcp(k, _):
                pltpu.sync_copy(r0.at[pl.ds(0, L)],
                                acc.at[pl.ds(s * NPT + k * L, L)])
                return 0
            lax.fori_loop(0, NPT // L, zcp, 0)
            plsc.subcore_barrier()

            def scale(buf, nb):
                def rowf(r, _):
                    v16 = plsc.load_gather(nb, [_full16(r)])
                    for j in range(F // L):
                        buf[r, pl.ds(j * L, L)] = buf[r, pl.ds(j * L, L)] * v16
                    return 0
                lax.fori_loop(0, CH, rowf, 0, unroll=4)

            idx_fetch(0, 0)
            idx_fetch(1, 1)
            idx_wait(0)
            pltpu.async_copy(xc.at[sidx[0]], rows[0], gsem[0])

            def outer(i, _):
                for kk in range(LCM):
                    k = i * LCM + kk
                    j3 = kk % 3
                    j4 = kk % 4
                    pltpu.make_async_copy(xc.at[sidx[j4]], rows[j3],
                                          gsem[j3]).wait()
                    @pl.when(k + 1 < NCH)
                    def _():
                        idx_wait((j4 + 1) % 4)
                        pltpu.async_copy(xc.at[sidx[(j4 + 1) % 4]],
                                         rows[(j3 + 1) % 3],
                                         gsem[(j3 + 1) % 3])
                    @pl.when(k + 2 < NCH)
                    def _():
                        idx_fetch(k + 2, (j4 + 2) % 4)
                    scale(rows[j3], nbuf[j4])
                    @pl.when(k >= 1)
                    def _():
                        pltpu.make_async_copy(rows[(j3 + 2) % 3],
                                              acc.at[didx[(j4 + 3) % 4]],
                                              ssem[(j3 + 2) % 3]).wait()
                    pltpu.async_copy(rows[j3], acc.at[didx[j4]], ssem[j3],
                                     add=True)
                return 0
            lax.fori_loop(0, NCH // LCM, outer, 0)
            pltpu.make_async_copy(rows[(NCH - 1) % 3],
                                  acc.at[didx[(NCH - 1) % 4]],
                                  ssem[(NCH - 1) % 3]).wait()
            plsc.subcore_barrier()

            pltpu.sync_copy(acc.at[pl.ds(s * NPT, NPT)],
                            outc.at[pl.ds(s * NPT, NPT)])

        @pl.when(c == 0)
        def _():
            body(x0, out0)

        @pl.when(c == 1)
        def _():
            body(x1, out1)

    return agg1_kernel


def _make_agg2_kernel(E_pad, N_pad):
    NPT = N_pad // NS
    NCH = E_pad // (NS * NC) // CH

    @functools.partial(
        pl.kernel,
        out_type=(jax.ShapeDtypeStruct((2, N_pad), jnp.float32),
                  jax.ShapeDtypeStruct((2, N_pad), jnp.float32)),
        mesh=plsc.VectorSubcoreMesh(**_MESH),
        **_NOLAYOUT,
        scratch_types=[
            pltpu.VMEM_SHARED((N_pad,), jnp.float32),
            pltpu.VMEM_SHARED((N_pad,), jnp.float32),
            pltpu.VMEM((N_pad,), jnp.float32),
            pltpu.VMEM((N_pad,), jnp.float32),
            pltpu.VMEM((NPT,), jnp.float32),
            pltpu.VMEM((NCH, 1, CH), jnp.int32),
            pltpu.VMEM((NCH, 1, CH), jnp.int32),
            pltpu.VMEM((NCH, 1, CH), jnp.float32),
            pltpu.VMEM((NCH, 1, CH), jnp.float32),
            pltpu.VMEM((NCH, 1, CH), jnp.float32),
            pltpu.SemaphoreType.DMA,
        ],
    )
    def agg2_kernel(src3, dst3, norm3, h0, h1, pA, pB,
                    acc0, acc1, h0l, h1l, slcb, sl, dl, nl, q0, q1, sem):
        c = lax.axis_index("c")
        s = lax.axis_index("s")
        w = c * NS + s
        row0 = w * NCH
        z16 = jnp.zeros((L,), jnp.float32)

        pltpu.sync_copy(h0, h0l)
        pltpu.sync_copy(h1, h1l)
        pltpu.sync_copy(src3.at[pl.ds(row0, NCH)], sl)
        pltpu.sync_copy(dst3.at[pl.ds(row0, NCH)], dl)
        pltpu.sync_copy(norm3.at[pl.ds(row0, NCH)], nl)

        def zero_body(k, _):
            slcb[pl.ds(k * L, L)] = z16
            return 0
        lax.fori_loop(0, NPT // L, zero_body, 0)
        pltpu.sync_copy(slcb, acc0.at[pl.ds(s * NPT, NPT)])
        pltpu.sync_copy(slcb, acc1.at[pl.ds(s * NPT, NPT)])
        plsc.subcore_barrier()

        def chunk(k, _):
            for j in range(CH // L):
                slj = pl.ds(j * L, L)
                s16 = sl[k, 0, slj]
                n16 = nl[k, 0, slj]
                q0[k, 0, slj] = plsc.load_gather(h0l, [s16]) * n16
                q1[k, 0, slj] = plsc.load_gather(h1l, [s16]) * n16
            return 0
        lax.fori_loop(0, NCH, chunk, 0)

        GF = 6
        for g in range(0, NCH, GF):
            n = min(GF, NCH - g)
            for t in range(n):
                pltpu.async_copy(q0.at[g + t, 0], acc0.at[dl.at[g + t, 0]],
                                 sem, add=True)
                pltpu.async_copy(q1.at[g + t, 0], acc1.at[dl.at[g + t, 0]],
                                 sem, add=True)
            for t in range(n):
                pltpu.make_async_copy(q0.at[g + t, 0],
                                      acc0.at[dl.at[g + t, 0]], sem).wait()
                pltpu.make_async_copy(q1.at[g + t, 0],
                                      acc1.at[dl.at[g + t, 0]], sem).wait()
        plsc.subcore_barrier()

        @pl.when(c == 0)
        def _():
            pltpu.sync_copy(acc0.at[pl.ds(s * NPT, NPT)],
                            pA.at[0, pl.ds(s * NPT, NPT)])
            pltpu.sync_copy(acc1.at[pl.ds(s * NPT, NPT)],
                            pA.at[1, pl.ds(s * NPT, NPT)])

        @pl.when(c == 1)
        def _():
            pltpu.sync_copy(acc0.at[pl.ds(s * NPT, NPT)],
                            pB.at[0, pl.ds(s * NPT, NPT)])
            pltpu.sync_copy(acc1.at[pl.ds(s * NPT, NPT)],
                            pB.at[1, pl.ds(s * NPT, NPT)])

    return agg2_kernel


def _make_mlp_kernel(N_pad, D, H, OUT):
    BR = 256
    F = D // 2

    def body(a0, a1, w1, b1r, w2, o):
        w1v = w1[...]
        h = (jnp.dot(a0[...], w1v[:F, :], preferred_element_type=jnp.float32)
             + jnp.dot(a1[...], w1v[F:, :], preferred_element_type=jnp.float32)
             + b1r[...])
        h = jnp.maximum(h, 0.0)
        o[...] = jnp.dot(h, w2[...], preferred_element_type=jnp.float32)

    return pl.pallas_call(
        body,
        grid=(N_pad // BR,),
        in_specs=[
            pl.BlockSpec((BR, F), lambda i: (i, 0)),
            pl.BlockSpec((BR, F), lambda i: (i, 0)),
            pl.BlockSpec((D, H), lambda i: (0, 0)),
            pl.BlockSpec((1, H), lambda i: (0, 0)),
            pl.BlockSpec((H, OUT), lambda i: (0, 0)),
        ],
        out_specs=pl.BlockSpec((BR, OUT), lambda i: (i, 0)),
        out_shape=jax.ShapeDtypeStruct((N_pad, OUT), jnp.float32),
    )


def kernel(x, edge_index, edge_weight, W1, b1, W2, b2):
    N, D = x.shape
    E = edge_index.shape[1]
    H = W1.shape[1]
    OUT = W2.shape[1]

    N_pad = -(-N // (NS * L)) * (NS * L)
    E1 = E + N
    step = NS * NC * CH * 3
    E_pad = -(-E1 // step) * step

    src = edge_index[0].astype(jnp.int32)
    dst = edge_index[1].astype(jnp.int32)
    loop = jnp.arange(N, dtype=jnp.int32)
    padz = jnp.zeros((E_pad - E1,), jnp.int32)
    srcp = jnp.concatenate([src, loop, padz]).reshape(E_pad // CH, 1, CH)
    dstp = jnp.concatenate([dst, loop, padz]).reshape(E_pad // CH, 1, CH)
    ewp = jnp.concatenate([edge_weight.astype(jnp.float32),
                           jnp.ones((N,), jnp.float32),
                           jnp.zeros((E_pad - E1,), jnp.float32)]
                          ).reshape(E_pad // CH, 1, CH)

    xp = jnp.pad(x.astype(jnp.float32), ((0, N_pad - N), (0, 0)))
    x0 = xp[:, : D // 2]
    x1 = xp[:, D // 2:]

    norm = _make_norm_kernel(E_pad, N_pad)(srcp, dstp, ewp)
    agg0, agg1 = _make_agg1_kernel(E_pad, N_pad, D // 2)(srcp, dstp, norm,
                                                         x0, x1)
    hw2 = _make_mlp_kernel(N_pad, D, H, OUT)(agg0, agg1, W1,
                                             b1.reshape(1, H), W2)
    h0 = hw2[:, 0]
    h1 = hw2[:, 1]
    pA, pB = _make_agg2_kernel(E_pad, N_pad)(srcp, dstp, norm, h0, h1)
    q = pA + pB
    out = q[:, :N].T + b2
    return out
